# speedup vs baseline: 1.6696x; 1.6696x over previous
import functools

import jax
import jax.numpy as jnp
from jax import lax
from jax.experimental import pallas as pl
from jax.experimental.pallas import tpu as pltpu
from jax.experimental.pallas import tpu_sc as plsc

_NC = 2
_NS = 16
_NW = _NC * _NS


@functools.partial(jax.jit, static_argnames=("d_model", "chunk"))
def _sc_embed(idx, W_E, d_model, chunk):
    n_chunks = idx.shape[1]
    total = _NW * n_chunks * chunk
    mesh = plsc.VectorSubcoreMesh(core_axis_name="c", subcore_axis_name="s")

    @functools.partial(
        pl.kernel,
        out_type=jax.ShapeDtypeStruct((total, d_model), jnp.float32),
        mesh=mesh,
        scratch_types=[
            pltpu.VMEM((n_chunks, chunk), jnp.int32),
            pltpu.VMEM((chunk, d_model), jnp.float32),
            pltpu.VMEM((chunk, d_model), jnp.float32),
            pltpu.SemaphoreType.DMA,
            pltpu.SemaphoreType.DMA,
        ],
    )
    def k(idx_hbm, table_hbm, out_hbm, idx_v, buf0, buf1, sem0, sem1):
        wid = lax.axis_index("s") * _NC + lax.axis_index("c")
        base = wid * n_chunks * chunk
        pltpu.sync_copy(idx_hbm.at[wid], idx_v)

        def gather(g, buf, sem):
            return pltpu.make_async_copy(table_hbm.at[idx_v.at[g]], buf, sem)

        gather(0, buf0, sem0).start()

        def body(i, carry):
            g = i * 2
            gather(g + 1, buf1, sem1).start()
            gather(g, buf0, sem0).wait()
            pltpu.sync_copy(buf0, out_hbm.at[pl.ds(base + g * chunk, chunk)])

            @pl.when(g + 2 < n_chunks)
            def _():
                gather(g + 2, buf0, sem0).start()

            gather(g + 1, buf1, sem1).wait()
            pltpu.sync_copy(
                buf1, out_hbm.at[pl.ds(base + (g + 1) * chunk, chunk)]
            )
            return carry

        lax.fori_loop(0, n_chunks // 2, body, 0, unroll=False)

    return k(idx, W_E)


def kernel(tokens, W_E):
    B, T = tokens.shape
    V, D = W_E.shape
    total = B * T
    chunk = 32
    n_chunks = total // (_NW * chunk)
    idx = tokens.reshape(_NW, n_chunks, chunk).astype(jnp.int32)
    out = _sc_embed(idx, W_E, D, chunk)
    return out.reshape(B, T, D)
